# baseline (speedup 1.0000x reference)
import functools

import jax
import jax.numpy as jnp
from jax import lax
from jax.experimental import pallas as pl
from jax.experimental.pallas import tpu as pltpu
from jax.experimental.pallas import tpu_sc as plsc

N_NODES = 10000
N_EDGES = 320000
D = 128
DH = 64
D_EDGE = 16

NT = 16
C = 128
N_SLABS = 3
EPT = 6912
E_SLAB = NT * EPT
E_PAD = N_SLABS * E_SLAB
N_CHUNKS = EPT // C
ACC_ROWS = 10240
ROWS_PT = ACC_ROWS // NT
N_PAD_ROWS = ACC_ROWS - N_NODES


def _tables_body(x_ref, wa_ref, wb_ref, xa_ref, xb_ref):
    x = x_ref[...]
    xa = jnp.dot(x, wa_ref[...], preferred_element_type=jnp.float32)
    xb = jnp.dot(x, wb_ref[...], preferred_element_type=jnp.float32)
    xa_ref[0] = xa[:, :DH]
    xa_ref[1] = xa[:, DH:]
    xb_ref[0] = xb[:, :DH]
    xb_ref[1] = xb[:, DH:]


def _make_tables(x_pad, w1at, w1bt):
    return pl.pallas_call(
        _tables_body,
        out_shape=(
            jax.ShapeDtypeStruct((2, ACC_ROWS, DH), jnp.float32),
            jax.ShapeDtypeStruct((2, ACC_ROWS, DH), jnp.float32),
        ),
    )(x_pad, w1at, w1bt)


_EPR = D // D_EDGE
_PROWS_SLAB = E_SLAB // _EPR
_PB = _PROWS_SLAB // 8


def _ea_body(attr_ref, wbd_ref, b1_ref, ea_ref):
    v = (
        jnp.dot(attr_ref[...], wbd_ref[...], preferred_element_type=jnp.float32)
        + b1_ref[...]
    )
    ea_ref[...] = v.reshape(_PB * _EPR, D)


def _make_ea(attr_packed, wbd, b1bd):
    return pl.pallas_call(
        _ea_body,
        grid=(_PROWS_SLAB // _PB,),
        in_specs=[
            pl.BlockSpec((_PB, D), lambda i: (i, 0)),
            pl.BlockSpec((D, _EPR * D), lambda i: (0, 0)),
            pl.BlockSpec((1, _EPR * D), lambda i: (0, 0)),
        ],
        out_specs=pl.BlockSpec((_PB * _EPR, D), lambda i: (i, 0)),
        out_shape=jax.ShapeDtypeStruct((E_SLAB, D), jnp.float32),
    )(attr_packed, wbd, b1bd)


def _sc_body(
    xa_hbm, xb_hbm, ea_hbm, src_hbm, dst_hbm,
    acc_out, cnt_out,
    idx_src0, idx_src1, idx_dst0, idx_dst1, idx_dsta0, idx_dsta1,
    rows_a0, rows_a1, rows_b0, rows_b1, ea_buf0, ea_buf1,
    ones_buf, zero_cnt,
    acc_sh, cnt_sh,
    sem_a0, sem_a1, sem_b0, sem_b1, sem_e0, sem_e1,
):
    cid = lax.axis_index("c")
    sid = lax.axis_index("s")
    col_off = cid * DH
    row_off = cid * ACC_ROWS

    idx_src = (idx_src0, idx_src1)
    idx_dst = (idx_dst0, idx_dst1)
    idx_dsta = (idx_dsta0, idx_dsta1)
    rows_a = (rows_a0, rows_a1)
    rows_b = (rows_b0, rows_b1)
    ea_buf = (ea_buf0, ea_buf1)
    sem_a = (sem_a0, sem_a1)
    sem_b = (sem_b0, sem_b1)
    sem_e = (sem_e0, sem_e1)

    def fill_rows(i, _):
        for j in range(DH // 16):
            rows_a0[i, pl.ds(j * 16, 16)] = jnp.zeros((16,), jnp.float32)
        ones_buf[i, pl.ds(0, 16)] = jnp.ones((16,), jnp.float32)
        zero_cnt[i, pl.ds(0, 16)] = jnp.zeros((16,), jnp.float32)
        return 0

    lax.fori_loop(0, C, fill_rows, 0)

    for k in range(ROWS_PT // C):
        pltpu.sync_copy(rows_a0, acc_sh.at[pl.ds(sid * ROWS_PT + k * C, C)])
        pltpu.sync_copy(zero_cnt, cnt_sh.at[pl.ds(sid * ROWS_PT + k * C, C)])
    plsc.subcore_barrier()

    ebase = sid * EPT

    def issue(k, b):
        base = ebase + k * C
        pltpu.sync_copy(src_hbm.at[pl.ds(base, C)], idx_src[b])
        pltpu.sync_copy(dst_hbm.at[pl.ds(base, C)], idx_dst[b])
        for j in range(C // 16):
            s = pl.ds(j * 16, 16)
            idx_src[b][s] = idx_src[b][s] + row_off
            idx_dsta[b][s] = idx_dst[b][s] + row_off
        pltpu.async_copy(xa_hbm.at[idx_dsta[b]], rows_a[b], sem_a[b])
        pltpu.async_copy(xb_hbm.at[idx_src[b]], rows_b[b], sem_b[b])
        pltpu.async_copy(
            ea_hbm.at[pl.ds(base, C), pl.ds(col_off, DH)], ea_buf[b], sem_e[b]
        )

    def consume(b):
        pltpu.make_async_copy(xa_hbm.at[idx_dsta[b]], rows_a[b], sem_a[b]).wait()
        pltpu.make_async_copy(xb_hbm.at[idx_src[b]], rows_b[b], sem_b[b]).wait()
        pltpu.make_async_copy(
            ea_hbm.at[pl.ds(0, C), pl.ds(col_off, DH)], ea_buf[b], sem_e[b]
        ).wait()

        def row(i, _):
            for j in range(DH // 16):
                s = pl.ds(j * 16, 16)
                v = rows_a[b][i, s] + rows_b[b][i, s] + ea_buf[b][i, s]
                rows_a[b][i, s] = jnp.maximum(v, 0.0)
            return 0

        lax.fori_loop(0, C, row, 0)

        pltpu.sync_copy(rows_a[b], acc_sh.at[idx_dst[b]], add=True)

        @pl.when(cid == b)
        def _():
            pltpu.sync_copy(ones_buf, cnt_sh.at[idx_dst[b]], add=True)

    issue(0, 0)

    def pair(k2, _):
        k = 2 * k2

        @pl.when(k + 1 < N_CHUNKS)
        def _():
            issue(k + 1, 1)

        consume(0)

        @pl.when(k + 2 < N_CHUNKS)
        def _():
            issue(k + 2, 0)

        consume(1)
        return 0

    lax.fori_loop(0, N_CHUNKS // 2, pair, 0)
    plsc.subcore_barrier()

    r0 = sid * ROWS_PT
    pltpu.sync_copy(
        acc_sh.at[pl.ds(r0, ROWS_PT)],
        acc_out.at[pl.ds(r0, ROWS_PT), pl.ds(col_off, DH)],
    )
    pltpu.sync_copy(cnt_sh.at[pl.ds(r0, ROWS_PT)], cnt_out.at[cid, pl.ds(r0, ROWS_PT)])


def _make_sc(xa, xb, ea, src_p, dst_p):
    mesh = plsc.VectorSubcoreMesh(core_axis_name="c", subcore_axis_name="s")
    vm = pltpu.VMEM
    f32 = jnp.float32
    f = functools.partial(
        pl.kernel,
        compiler_params=pltpu.CompilerParams(use_tc_tiling_on_sc=False),
        out_type=(
            jax.ShapeDtypeStruct((ACC_ROWS, D), f32),
            jax.ShapeDtypeStruct((2, ACC_ROWS, 16), f32),
        ),
        mesh=mesh,
        scratch_types=[
            vm((C,), jnp.int32), vm((C,), jnp.int32),
            vm((C,), jnp.int32), vm((C,), jnp.int32),
            vm((C,), jnp.int32), vm((C,), jnp.int32),
            vm((C, DH), f32), vm((C, DH), f32),
            vm((C, DH), f32), vm((C, DH), f32),
            vm((C, DH), f32), vm((C, DH), f32),
            vm((C, 16), f32),
            vm((C, 16), f32),
            pltpu.VMEM_SHARED((ACC_ROWS, DH), f32),
            pltpu.VMEM_SHARED((ACC_ROWS, 16), f32),
            pltpu.SemaphoreType.DMA, pltpu.SemaphoreType.DMA,
            pltpu.SemaphoreType.DMA, pltpu.SemaphoreType.DMA,
            pltpu.SemaphoreType.DMA, pltpu.SemaphoreType.DMA,
        ],
    )(_sc_body)
    return f(xa, xb, ea, src_p, dst_p)


def _finish_body(
    acc0_ref, acc1_ref, acc2_ref, cnt0_ref, cnt1_ref, cnt2_ref,
    w2t_ref, b2_ref, out_ref,
):
    s = acc0_ref[:N_NODES, :] + acc1_ref[:N_NODES, :] + acc2_ref[:N_NODES, :]
    c = (
        cnt0_ref[0, :N_NODES, 0:1] + cnt0_ref[1, :N_NODES, 0:1]
        + cnt1_ref[0, :N_NODES, 0:1] + cnt1_ref[1, :N_NODES, 0:1]
        + cnt2_ref[0, :N_NODES, 0:1] + cnt2_ref[1, :N_NODES, 0:1]
    )
    c = jnp.maximum(c, 1.0)
    out_ref[...] = (
        jnp.dot(s / c, w2t_ref[...], preferred_element_type=jnp.float32)
        + b2_ref[...]
    )


def _make_finish(accs, cnts, w2t, b2r):
    return pl.pallas_call(
        _finish_body,
        out_shape=jax.ShapeDtypeStruct((N_NODES, D), jnp.float32),
    )(*accs, *cnts, w2t, b2r)


@jax.jit
def kernel(x, edge_index, edge_attr, W1, b1, W2, b2):
    src = edge_index[0].astype(jnp.int32)
    dst = edge_index[1].astype(jnp.int32)

    n_pad = E_PAD - N_EDGES
    pad_idx = (N_NODES + jnp.arange(n_pad, dtype=jnp.int32) % N_PAD_ROWS)
    src_p = jnp.concatenate([src, pad_idx])
    dst_p = jnp.concatenate([dst, pad_idx])
    x_pad = jnp.concatenate(
        [x, jnp.zeros((ACC_ROWS - N_NODES, x.shape[1]), jnp.float32)]
    )

    w1at = W1[:, :D].T
    w1bt = W1[:, D:2 * D].T
    w1ct = W1[:, 2 * D:].T
    wbd = jnp.kron(jnp.eye(_EPR, dtype=jnp.float32), w1ct)
    b1bd = jnp.tile(b1, _EPR).reshape(1, _EPR * D)
    b2r = b2.reshape(1, D)

    xa, xb = _make_tables(x_pad, w1at, w1bt)
    xa = xa.reshape(2 * ACC_ROWS, DH)
    xb = xb.reshape(2 * ACC_ROWS, DH)

    accs, cnts = [], []
    for s in range(N_SLABS):
        lo, hi = s * E_SLAB, (s + 1) * E_SLAB
        if hi <= N_EDGES:
            attr_p = edge_attr[lo:hi].reshape(_PROWS_SLAB, D)
        else:
            attr_p = jnp.concatenate(
                [
                    edge_attr[lo:].reshape((N_EDGES - lo) // _EPR, D),
                    jnp.zeros(((hi - N_EDGES) // _EPR, D), jnp.float32),
                ]
            )
        ea_s = _make_ea(attr_p, wbd, b1bd)
        acc_s, cnt_s = _make_sc(xa, xb, ea_s, src_p[lo:hi], dst_p[lo:hi])
        accs.append(acc_s)
        cnts.append(cnt_s)
    return _make_finish(accs, cnts, W2.T, b2r)


# speedup vs baseline: 1.0348x; 1.0348x over previous
import functools

import jax
import jax.numpy as jnp
from jax import lax
from jax.experimental import pallas as pl
from jax.experimental.pallas import tpu as pltpu
from jax.experimental.pallas import tpu_sc as plsc

N_NODES = 10000
N_EDGES = 320000
D = 128
DH = 64
D_EDGE = 16

NT = 16
C = 128
N_SLABS = 2
EPT = 10240
E_SLAB = NT * EPT
E_PAD = N_SLABS * E_SLAB
N_CHUNKS = EPT // C
ACC_ROWS = 10240
ROWS_PT = ACC_ROWS // NT
N_PAD_ROWS = ACC_ROWS - N_NODES


def _tables_body(x_ref, wa_ref, wb_ref, xa_ref, xb_ref):
    x = x_ref[...]
    xa = jnp.dot(x, wa_ref[...], preferred_element_type=jnp.float32)
    xb = jnp.dot(x, wb_ref[...], preferred_element_type=jnp.float32)
    xa_ref[0] = xa[:, :DH]
    xa_ref[1] = xa[:, DH:]
    xb_ref[0] = xb[:, :DH]
    xb_ref[1] = xb[:, DH:]


def _make_tables(x_pad, w1at, w1bt):
    return pl.pallas_call(
        _tables_body,
        out_shape=(
            jax.ShapeDtypeStruct((2, ACC_ROWS, DH), jnp.float32),
            jax.ShapeDtypeStruct((2, ACC_ROWS, DH), jnp.float32),
        ),
    )(x_pad, w1at, w1bt)


_EPR = D // D_EDGE
_PROWS_SLAB = E_SLAB // _EPR
_PB = _PROWS_SLAB // 8


def _ea_body(attr_ref, wbd_ref, b1_ref, ea_ref):
    v = (
        jnp.dot(attr_ref[...], wbd_ref[...], preferred_element_type=jnp.float32)
        + b1_ref[...]
    )
    ea_ref[...] = v.reshape(_PB * _EPR, D)


def _make_ea(attr_packed, wbd, b1bd):
    return pl.pallas_call(
        _ea_body,
        grid=(_PROWS_SLAB // _PB,),
        in_specs=[
            pl.BlockSpec((_PB, D), lambda i: (i, 0)),
            pl.BlockSpec((D, _EPR * D), lambda i: (0, 0)),
            pl.BlockSpec((1, _EPR * D), lambda i: (0, 0)),
        ],
        out_specs=pl.BlockSpec((_PB * _EPR, D), lambda i: (i, 0)),
        out_shape=jax.ShapeDtypeStruct((E_SLAB, D), jnp.float32),
    )(attr_packed, wbd, b1bd)


def _sc_body(
    xa_hbm, xb_hbm, ea_hbm, src_hbm, dst_hbm,
    acc_out, cnt_out,
    idx_src0, idx_src1, idx_dst0, idx_dst1, idx_dsta0, idx_dsta1,
    rows_a0, rows_a1, rows_b0, rows_b1, ea_buf0, ea_buf1,
    ones_buf, zero_cnt,
    acc_sh, cnt_sh,
    sem_a0, sem_a1, sem_b0, sem_b1, sem_e0, sem_e1,
):
    cid = lax.axis_index("c")
    sid = lax.axis_index("s")
    col_off = cid * DH
    row_off = cid * ACC_ROWS

    idx_src = (idx_src0, idx_src1)
    idx_dst = (idx_dst0, idx_dst1)
    idx_dsta = (idx_dsta0, idx_dsta1)
    rows_a = (rows_a0, rows_a1)
    rows_b = (rows_b0, rows_b1)
    ea_buf = (ea_buf0, ea_buf1)
    sem_a = (sem_a0, sem_a1)
    sem_b = (sem_b0, sem_b1)
    sem_e = (sem_e0, sem_e1)

    def fill_rows(i, _):
        for j in range(DH // 16):
            rows_a0[i, pl.ds(j * 16, 16)] = jnp.zeros((16,), jnp.float32)
        ones_buf[i, pl.ds(0, 16)] = jnp.ones((16,), jnp.float32)
        zero_cnt[i, pl.ds(0, 16)] = jnp.zeros((16,), jnp.float32)
        return 0

    lax.fori_loop(0, C, fill_rows, 0)

    for k in range(ROWS_PT // C):
        pltpu.sync_copy(rows_a0, acc_sh.at[pl.ds(sid * ROWS_PT + k * C, C)])
        pltpu.sync_copy(zero_cnt, cnt_sh.at[pl.ds(sid * ROWS_PT + k * C, C)])
    plsc.subcore_barrier()

    ebase = sid * EPT

    def issue(k, b):
        base = ebase + k * C
        pltpu.sync_copy(src_hbm.at[pl.ds(base, C)], idx_src[b])
        pltpu.sync_copy(dst_hbm.at[pl.ds(base, C)], idx_dst[b])
        for j in range(C // 16):
            s = pl.ds(j * 16, 16)
            idx_src[b][s] = idx_src[b][s] + row_off
            idx_dsta[b][s] = idx_dst[b][s] + row_off
        pltpu.async_copy(xa_hbm.at[idx_dsta[b]], rows_a[b], sem_a[b])
        pltpu.async_copy(xb_hbm.at[idx_src[b]], rows_b[b], sem_b[b])
        pltpu.async_copy(
            ea_hbm.at[pl.ds(base, C), pl.ds(col_off, DH)], ea_buf[b], sem_e[b]
        )

    def consume(b):
        pltpu.make_async_copy(xa_hbm.at[idx_dsta[b]], rows_a[b], sem_a[b]).wait()
        pltpu.make_async_copy(xb_hbm.at[idx_src[b]], rows_b[b], sem_b[b]).wait()
        pltpu.make_async_copy(
            ea_hbm.at[pl.ds(0, C), pl.ds(col_off, DH)], ea_buf[b], sem_e[b]
        ).wait()

        def row(i, _):
            for j in range(DH // 16):
                s = pl.ds(j * 16, 16)
                v = rows_a[b][i, s] + rows_b[b][i, s] + ea_buf[b][i, s]
                rows_a[b][i, s] = jnp.maximum(v, 0.0)
            return 0

        lax.fori_loop(0, C, row, 0)

        pltpu.sync_copy(rows_a[b], acc_sh.at[idx_dst[b]], add=True)

        @pl.when(cid == b)
        def _():
            pltpu.sync_copy(ones_buf, cnt_sh.at[idx_dst[b]], add=True)

    issue(0, 0)

    def pair(k2, _):
        k = 2 * k2

        @pl.when(k + 1 < N_CHUNKS)
        def _():
            issue(k + 1, 1)

        consume(0)

        @pl.when(k + 2 < N_CHUNKS)
        def _():
            issue(k + 2, 0)

        consume(1)
        return 0

    lax.fori_loop(0, N_CHUNKS // 2, pair, 0)
    plsc.subcore_barrier()

    r0 = sid * ROWS_PT
    pltpu.sync_copy(
        acc_sh.at[pl.ds(r0, ROWS_PT)],
        acc_out.at[pl.ds(r0, ROWS_PT), pl.ds(col_off, DH)],
    )
    pltpu.sync_copy(cnt_sh.at[pl.ds(r0, ROWS_PT)], cnt_out.at[cid, pl.ds(r0, ROWS_PT)])


def _make_sc(xa, xb, ea, src_p, dst_p):
    mesh = plsc.VectorSubcoreMesh(core_axis_name="c", subcore_axis_name="s")
    vm = pltpu.VMEM
    f32 = jnp.float32
    f = functools.partial(
        pl.kernel,
        compiler_params=pltpu.CompilerParams(use_tc_tiling_on_sc=False),
        out_type=(
            jax.ShapeDtypeStruct((ACC_ROWS, D), f32),
            jax.ShapeDtypeStruct((2, ACC_ROWS, 16), f32),
        ),
        mesh=mesh,
        scratch_types=[
            vm((C,), jnp.int32), vm((C,), jnp.int32),
            vm((C,), jnp.int32), vm((C,), jnp.int32),
            vm((C,), jnp.int32), vm((C,), jnp.int32),
            vm((C, DH), f32), vm((C, DH), f32),
            vm((C, DH), f32), vm((C, DH), f32),
            vm((C, DH), f32), vm((C, DH), f32),
            vm((C, 16), f32),
            vm((C, 16), f32),
            pltpu.VMEM_SHARED((ACC_ROWS, DH), f32),
            pltpu.VMEM_SHARED((ACC_ROWS, 16), f32),
            pltpu.SemaphoreType.DMA, pltpu.SemaphoreType.DMA,
            pltpu.SemaphoreType.DMA, pltpu.SemaphoreType.DMA,
            pltpu.SemaphoreType.DMA, pltpu.SemaphoreType.DMA,
        ],
    )(_sc_body)
    return f(xa, xb, ea, src_p, dst_p)


def _make_finish(accs, cnts, w2t, b2r):
    ns = len(accs)

    def body(*refs):
        acc_refs = refs[:ns]
        cnt_refs = refs[ns:2 * ns]
        w2t_ref, b2_ref, out_ref = refs[2 * ns:]
        s = acc_refs[0][:N_NODES, :]
        for a in acc_refs[1:]:
            s = s + a[:N_NODES, :]
        c = jnp.zeros((N_NODES, 1), jnp.float32)
        for cr in cnt_refs:
            c = c + cr[0, :N_NODES, 0:1] + cr[1, :N_NODES, 0:1]
        c = jnp.maximum(c, 1.0)
        out_ref[...] = (
            jnp.dot(s / c, w2t_ref[...], preferred_element_type=jnp.float32)
            + b2_ref[...]
        )

    return pl.pallas_call(
        body,
        out_shape=jax.ShapeDtypeStruct((N_NODES, D), jnp.float32),
    )(*accs, *cnts, w2t, b2r)


@jax.jit
def kernel(x, edge_index, edge_attr, W1, b1, W2, b2):
    src = edge_index[0].astype(jnp.int32)
    dst = edge_index[1].astype(jnp.int32)

    n_pad = E_PAD - N_EDGES
    pad_idx = (N_NODES + jnp.arange(n_pad, dtype=jnp.int32) % N_PAD_ROWS)
    src_p = jnp.concatenate([src, pad_idx])
    dst_p = jnp.concatenate([dst, pad_idx])
    x_pad = jnp.concatenate(
        [x, jnp.zeros((ACC_ROWS - N_NODES, x.shape[1]), jnp.float32)]
    )

    w1at = W1[:, :D].T
    w1bt = W1[:, D:2 * D].T
    w1ct = W1[:, 2 * D:].T
    wbd = jnp.kron(jnp.eye(_EPR, dtype=jnp.float32), w1ct)
    b1bd = jnp.tile(b1, _EPR).reshape(1, _EPR * D)
    b2r = b2.reshape(1, D)

    xa, xb = _make_tables(x_pad, w1at, w1bt)
    xa = xa.reshape(2 * ACC_ROWS, DH)
    xb = xb.reshape(2 * ACC_ROWS, DH)

    accs, cnts = [], []
    for s in range(N_SLABS):
        lo, hi = s * E_SLAB, (s + 1) * E_SLAB
        if hi <= N_EDGES:
            attr_p = edge_attr[lo:hi].reshape(_PROWS_SLAB, D)
        else:
            attr_p = jnp.concatenate(
                [
                    edge_attr[lo:].reshape((N_EDGES - lo) // _EPR, D),
                    jnp.zeros(((hi - N_EDGES) // _EPR, D), jnp.float32),
                ]
            )
        ea_s = _make_ea(attr_p, wbd, b1bd)
        acc_s, cnt_s = _make_sc(xa, xb, ea_s, src_p[lo:hi], dst_p[lo:hi])
        accs.append(acc_s)
        cnts.append(cnt_s)
    return _make_finish(accs, cnts, W2.T, b2r)


# speedup vs baseline: 1.0822x; 1.0458x over previous
import functools

import jax
import jax.numpy as jnp
from jax import lax
from jax.experimental import pallas as pl
from jax.experimental.pallas import tpu as pltpu
from jax.experimental.pallas import tpu_sc as plsc

N_NODES = 10000
N_EDGES = 320000
D = 128
DH = 64
D_EDGE = 16

NT = 16
C = 128
N_SLABS = 2
EPT = 10240
E_SLAB = NT * EPT
E_PAD = N_SLABS * E_SLAB
N_CHUNKS = EPT // C
ACC_ROWS = 10240
ROWS_PT = ACC_ROWS // NT
N_PAD_ROWS = ACC_ROWS - N_NODES


def _tables_body(x_ref, wa_ref, wb_ref, xa_ref, xb_ref):
    x = x_ref[...]
    xa = jnp.dot(x, wa_ref[...], preferred_element_type=jnp.float32)
    xb = jnp.dot(x, wb_ref[...], preferred_element_type=jnp.float32)
    xa_ref[0] = xa[:, :DH]
    xa_ref[1] = xa[:, DH:]
    xb_ref[0] = xb[:, :DH]
    xb_ref[1] = xb[:, DH:]


def _make_tables(x_pad, w1at, w1bt):
    return pl.pallas_call(
        _tables_body,
        out_shape=(
            jax.ShapeDtypeStruct((2, ACC_ROWS, DH), jnp.float32),
            jax.ShapeDtypeStruct((2, ACC_ROWS, DH), jnp.float32),
        ),
    )(x_pad, w1at, w1bt)


_EPR = D // D_EDGE
_PROWS_SLAB = E_SLAB // _EPR
_PB = _PROWS_SLAB // 8


def _ea_body(attr_ref, wbd_ref, b1_ref, ea_ref, *, pb):
    v = (
        jnp.dot(attr_ref[...], wbd_ref[...], preferred_element_type=jnp.float32)
        + b1_ref[...]
    )
    ea_ref[...] = v.reshape(pb * _EPR, D)


def _make_ea(attr_packed, wbd, b1bd):
    prows = attr_packed.shape[0]
    pb = prows // 8
    body = functools.partial(_ea_body, pb=pb)
    return pl.pallas_call(
        body,
        grid=(prows // pb,),
        in_specs=[
            pl.BlockSpec((pb, D), lambda i: (i, 0)),
            pl.BlockSpec((D, _EPR * D), lambda i: (0, 0)),
            pl.BlockSpec((1, _EPR * D), lambda i: (0, 0)),
        ],
        out_specs=pl.BlockSpec((pb * _EPR, D), lambda i: (i, 0)),
        out_shape=jax.ShapeDtypeStruct((prows * _EPR, D), jnp.float32),
    )(attr_packed, wbd, b1bd)


def _sc_body(
    xa_hbm, xb_hbm, ea_hbm, src_hbm, dst_hbm,
    acc_out, cnt_out,
    idx_src0, idx_src1, idx_dst0, idx_dst1, idx_dsta0, idx_dsta1,
    rows_a0, rows_a1, rows_b0, rows_b1, ea_buf0, ea_buf1,
    ones_buf, zero_cnt,
    acc_sh, cnt_sh,
    sem_a0, sem_a1, sem_b0, sem_b1, sem_e0, sem_e1,
    *, ept, n_chunks,
):
    cid = lax.axis_index("c")
    sid = lax.axis_index("s")
    col_off = cid * DH
    row_off = cid * ACC_ROWS

    idx_src = (idx_src0, idx_src1)
    idx_dst = (idx_dst0, idx_dst1)
    idx_dsta = (idx_dsta0, idx_dsta1)
    rows_a = (rows_a0, rows_a1)
    rows_b = (rows_b0, rows_b1)
    ea_buf = (ea_buf0, ea_buf1)
    sem_a = (sem_a0, sem_a1)
    sem_b = (sem_b0, sem_b1)
    sem_e = (sem_e0, sem_e1)

    def fill_rows(i, _):
        for j in range(DH // 16):
            rows_a0[i, pl.ds(j * 16, 16)] = jnp.zeros((16,), jnp.float32)
        ones_buf[i, pl.ds(0, 16)] = jnp.ones((16,), jnp.float32)
        zero_cnt[i, pl.ds(0, 16)] = jnp.zeros((16,), jnp.float32)
        return 0

    lax.fori_loop(0, C, fill_rows, 0)

    for k in range(ROWS_PT // C):
        pltpu.sync_copy(rows_a0, acc_sh.at[pl.ds(sid * ROWS_PT + k * C, C)])
        pltpu.sync_copy(zero_cnt, cnt_sh.at[pl.ds(sid * ROWS_PT + k * C, C)])
    plsc.subcore_barrier()

    ebase = sid * ept

    def issue(k, b):
        base = ebase + k * C
        pltpu.sync_copy(src_hbm.at[pl.ds(base, C)], idx_src[b])
        pltpu.sync_copy(dst_hbm.at[pl.ds(base, C)], idx_dst[b])
        for j in range(C // 16):
            s = pl.ds(j * 16, 16)
            idx_src[b][s] = idx_src[b][s] + row_off
            idx_dsta[b][s] = idx_dst[b][s] + row_off
        pltpu.async_copy(xa_hbm.at[idx_dsta[b]], rows_a[b], sem_a[b])
        pltpu.async_copy(xb_hbm.at[idx_src[b]], rows_b[b], sem_b[b])
        pltpu.async_copy(
            ea_hbm.at[pl.ds(base, C), pl.ds(col_off, DH)], ea_buf[b], sem_e[b]
        )

    def consume(b):
        pltpu.make_async_copy(xa_hbm.at[idx_dsta[b]], rows_a[b], sem_a[b]).wait()
        pltpu.make_async_copy(xb_hbm.at[idx_src[b]], rows_b[b], sem_b[b]).wait()
        pltpu.make_async_copy(
            ea_hbm.at[pl.ds(0, C), pl.ds(col_off, DH)], ea_buf[b], sem_e[b]
        ).wait()

        def row(i, _):
            for j in range(DH // 16):
                s = pl.ds(j * 16, 16)
                v = rows_a[b][i, s] + rows_b[b][i, s] + ea_buf[b][i, s]
                rows_a[b][i, s] = jnp.maximum(v, 0.0)
            return 0

        lax.fori_loop(0, C, row, 0)

        pltpu.sync_copy(rows_a[b], acc_sh.at[idx_dst[b]], add=True)

        @pl.when(cid == b)
        def _():
            pltpu.sync_copy(ones_buf, cnt_sh.at[idx_dst[b]], add=True)

    issue(0, 0)

    def pair(k2, _):
        k = 2 * k2

        @pl.when(k + 1 < n_chunks)
        def _():
            issue(k + 1, 1)

        consume(0)

        @pl.when(k + 2 < n_chunks)
        def _():
            issue(k + 2, 0)

        consume(1)
        return 0

    lax.fori_loop(0, n_chunks // 2, pair, 0)
    plsc.subcore_barrier()

    r0 = sid * ROWS_PT
    pltpu.sync_copy(
        acc_sh.at[pl.ds(r0, ROWS_PT)],
        acc_out.at[pl.ds(r0, ROWS_PT), pl.ds(col_off, DH)],
    )
    pltpu.sync_copy(cnt_sh.at[pl.ds(r0, ROWS_PT)], cnt_out.at[cid, pl.ds(r0, ROWS_PT)])


def _make_sc(xa, xb, ea, src_p, dst_p):
    mesh = plsc.VectorSubcoreMesh(core_axis_name="c", subcore_axis_name="s")
    vm = pltpu.VMEM
    f32 = jnp.float32
    ept = src_p.shape[0] // NT
    body = functools.partial(_sc_body, ept=ept, n_chunks=ept // C)
    f = functools.partial(
        pl.kernel,
        compiler_params=pltpu.CompilerParams(use_tc_tiling_on_sc=False),
        out_type=(
            jax.ShapeDtypeStruct((ACC_ROWS, D), f32),
            jax.ShapeDtypeStruct((2, ACC_ROWS, 16), f32),
        ),
        mesh=mesh,
        scratch_types=[
            vm((C,), jnp.int32), vm((C,), jnp.int32),
            vm((C,), jnp.int32), vm((C,), jnp.int32),
            vm((C,), jnp.int32), vm((C,), jnp.int32),
            vm((C, DH), f32), vm((C, DH), f32),
            vm((C, DH), f32), vm((C, DH), f32),
            vm((C, DH), f32), vm((C, DH), f32),
            vm((C, 16), f32),
            vm((C, 16), f32),
            pltpu.VMEM_SHARED((ACC_ROWS, DH), f32),
            pltpu.VMEM_SHARED((ACC_ROWS, 16), f32),
            pltpu.SemaphoreType.DMA, pltpu.SemaphoreType.DMA,
            pltpu.SemaphoreType.DMA, pltpu.SemaphoreType.DMA,
            pltpu.SemaphoreType.DMA, pltpu.SemaphoreType.DMA,
        ],
    )(body)
    return f(xa, xb, ea, src_p, dst_p)


def _make_finish(accs, cnts, w2t, b2r):
    ns = len(accs)

    def body(*refs):
        acc_refs = refs[:ns]
        cnt_refs = refs[ns:2 * ns]
        w2t_ref, b2_ref, out_ref = refs[2 * ns:]
        s = acc_refs[0][:N_NODES, :]
        for a in acc_refs[1:]:
            s = s + a[:N_NODES, :]
        c = jnp.zeros((N_NODES, 1), jnp.float32)
        for cr in cnt_refs:
            c = c + cr[0, :N_NODES, 0:1] + cr[1, :N_NODES, 0:1]
        c = jnp.maximum(c, 1.0)
        out_ref[...] = (
            jnp.dot(s / c, w2t_ref[...], preferred_element_type=jnp.float32)
            + b2_ref[...]
        )

    return pl.pallas_call(
        body,
        out_shape=jax.ShapeDtypeStruct((N_NODES, D), jnp.float32),
    )(*accs, *cnts, w2t, b2r)


@jax.jit
def kernel(x, edge_index, edge_attr, W1, b1, W2, b2):
    src = edge_index[0].astype(jnp.int32)
    dst = edge_index[1].astype(jnp.int32)

    n_pad = E_PAD - N_EDGES
    pad_idx = (N_NODES + jnp.arange(n_pad, dtype=jnp.int32) % N_PAD_ROWS)
    src_p = jnp.concatenate([src, pad_idx])
    dst_p = jnp.concatenate([dst, pad_idx])
    x_pad = jnp.concatenate(
        [x, jnp.zeros((ACC_ROWS - N_NODES, x.shape[1]), jnp.float32)]
    )

    w1at = W1[:, :D].T
    w1bt = W1[:, D:2 * D].T
    w1ct = W1[:, 2 * D:].T
    wbd = jnp.kron(jnp.eye(_EPR, dtype=jnp.float32), w1ct)
    b1bd = jnp.tile(b1, _EPR).reshape(1, _EPR * D)
    b2r = b2.reshape(1, D)

    xa, xb = _make_tables(x_pad, w1at, w1bt)
    xa = xa.reshape(2 * ACC_ROWS, DH)
    xb = xb.reshape(2 * ACC_ROWS, DH)

    slab_sizes = (106496, 221184)
    accs, cnts = [], []
    lo = 0
    for size in slab_sizes:
        hi = lo + size
        if hi <= N_EDGES:
            attr_p = edge_attr[lo:hi].reshape(size // _EPR, D)
        else:
            attr_p = jnp.concatenate(
                [
                    edge_attr[lo:].reshape((N_EDGES - lo) // _EPR, D),
                    jnp.zeros(((hi - N_EDGES) // _EPR, D), jnp.float32),
                ]
            )
        ea_s = _make_ea(attr_p, wbd, b1bd)
        acc_s, cnt_s = _make_sc(xa, xb, ea_s, src_p[lo:hi], dst_p[lo:hi])
        accs.append(acc_s)
        cnts.append(cnt_s)
        lo = hi
    return _make_finish(accs, cnts, W2.T, b2r)
